# Initial kernel scaffold; baseline (speedup 1.0000x reference)
import jax
import jax.numpy as jnp
from jax.experimental import pallas as pl


def kernel(w1s, b1s, w2s, b2s, wf1, bf1, wf2, bf2, x):
    raise NotImplementedError("write your pallas kernel here")


# speedup vs baseline: 3.2396x; 3.2396x over previous
import jax
import jax.numpy as jnp
from jax.experimental import pallas as pl
from jax.experimental.pallas import tpu as pltpu


def _band(v, out_len, in_len):
    k = v.shape[-1]
    u = jnp.pad(v, [(0, 0)] * (v.ndim - 1) + [(0, in_len + 1 - k)])
    t = jnp.tile(u, (1,) * (v.ndim - 1) + (out_len,))
    t = t[..., : out_len * in_len]
    return t.reshape(v.shape[:-1] + (out_len, in_len))


def _net2_body(xt_ref, w1b_ref, w2b_ref, wf1_ref, wf2_ref,
               b1_ref, b2_ref, bf1_ref, bf2_ref, out_ref, p1_ref):
    TB = xt_ref.shape[-1]
    f32 = jnp.float32

    for g in range(3):
        xg = xt_ref[pl.ds(g * 224, 336), :]
        og = jnp.dot(w1b_ref[...], xg, preferred_element_type=f32)
        v = og.reshape(4, 2, 96, TB)
        m = jnp.maximum(v[:, 0], v[:, 1]).reshape(4, 2, 48, TB)
        m = jnp.maximum(m[:, 0], m[:, 1])
        m = jnp.maximum(m + b1_ref[...], 0.0)
        p1_ref[pl.ds(g * 192, 192), :] = m.reshape(192, TB).astype(jnp.bfloat16)

    o2 = jnp.dot(w2b_ref[...], p1_ref[...], preferred_element_type=f32)
    v = o2.reshape(4, 2, 80, TB)
    m = jnp.maximum(v[:, 0], v[:, 1]).reshape(4, 2, 40, TB)
    m = jnp.maximum(m[:, 0], m[:, 1])
    p2 = jnp.maximum(m + b2_ref[...], 0.0).reshape(160, TB).astype(jnp.bfloat16)

    h1 = jnp.dot(wf1_ref[...], p2, preferred_element_type=f32) + bf1_ref[...]
    h1 = jnp.maximum(h1, 0.0).astype(jnp.bfloat16)
    z = jnp.dot(wf2_ref[...], h1, preferred_element_type=f32) + bf2_ref[...]

    zm = jnp.max(z, axis=0, keepdims=True)
    s = z - zm
    lse = jnp.log(jnp.sum(jnp.exp(s), axis=0, keepdims=True))
    out_ref[...] = s - lse


@jax.jit
def _net2(w1s, b1s, w2s, b2s, wf1, bf1, wf2, bf2, x):
    B = x.shape[0]
    TB = 256 if B % 256 == 0 else (128 if B % 128 == 0 else B)
    f32 = jnp.float32
    bf16 = jnp.bfloat16

    w1r = w1s.reshape(4, 5, 5).astype(f32)
    a = _band(w1r, 24, 28)
    a = a.transpose(0, 2, 3, 1)
    b = _band(a, 8, 12)
    b = b.reshape(4, 12, 2, 28, 8, 12)
    w1b = b.transpose(4, 2, 1, 0, 5, 3).reshape(768, 336).astype(bf16)

    w2r = w2s.reshape(10, 4, 5, 5).astype(f32)
    c = _band(w2r, 8, 12)
    c = c.transpose(0, 1, 3, 4, 2)
    d = _band(c, 8, 12)
    d = d.reshape(10, 4, 4, 2, 12, 8, 12)
    w2b = d.transpose(5, 3, 2, 0, 6, 4, 1).reshape(640, 576).astype(bf16)

    wf1p = wf1.reshape(100, 10, 4, 4).transpose(0, 2, 3, 1).reshape(100, 160)
    wf1p = wf1p.astype(bf16)
    wf2b = wf2.astype(bf16)

    b1bc = jnp.broadcast_to(jnp.tile(b1s.astype(f32), 12)[:, None], (48, TB))
    b2bc = jnp.broadcast_to(jnp.tile(b2s.astype(f32), 4)[:, None], (40, TB))
    bf1bc = jnp.broadcast_to(bf1.astype(f32).reshape(100, 1), (100, TB))
    bf2bc = jnp.broadcast_to(bf2.astype(f32).reshape(10, 1), (10, TB))

    xt = x.reshape(B, 784).astype(bf16).T

    out = pl.pallas_call(
        _net2_body,
        out_shape=jax.ShapeDtypeStruct((10, B), f32),
        grid=(B // TB,),
        in_specs=[
            pl.BlockSpec((784, TB), lambda i: (0, i)),
            pl.BlockSpec((768, 336), lambda i: (0, 0)),
            pl.BlockSpec((640, 576), lambda i: (0, 0)),
            pl.BlockSpec((100, 160), lambda i: (0, 0)),
            pl.BlockSpec((10, 100), lambda i: (0, 0)),
            pl.BlockSpec((48, TB), lambda i: (0, 0)),
            pl.BlockSpec((40, TB), lambda i: (0, 0)),
            pl.BlockSpec((100, TB), lambda i: (0, 0)),
            pl.BlockSpec((10, TB), lambda i: (0, 0)),
        ],
        out_specs=pl.BlockSpec((10, TB), lambda i: (0, i)),
        scratch_shapes=[pltpu.VMEM((576, TB), jnp.bfloat16)],
        compiler_params=pltpu.CompilerParams(
            dimension_semantics=("parallel",)),
    )(xt, w1b, w2b, wf1p, wf2b, b1bc, b2bc, bf1bc, bf2bc)

    return out.T


def kernel(w1s, b1s, w2s, b2s, wf1, bf1, wf2, bf2, x):
    return _net2(w1s, b1s, w2s, b2s, wf1, bf1, wf2, bf2, x)
